# baseline (speedup 1.0000x reference)
import jax
import jax.numpy as jnp
from jax.experimental import pallas as pl
from jax.experimental.pallas import tpu as pltpu

B = 32
D = 1024
V = 100000
BV = 2048
NB = (V + BV - 1) // BV


def _top1_body(x_ref, w_ref, b_ref, out_ref, best_val, best_idx):
    i = pl.program_id(0)
    logits = jnp.dot(x_ref[...], w_ref[...],
                     preferred_element_type=jnp.float32) + b_ref[...]
    col = jax.lax.broadcasted_iota(jnp.int32, (1, BV), 1) + i * BV
    logits = jnp.where(col < V, logits, -jnp.inf)
    lmax = jnp.max(logits)
    lidx = jnp.min(jnp.where(logits == lmax, col, V))

    @pl.when(i == 0)
    def _():
        best_val[0] = lmax
        best_idx[0] = lidx

    @pl.when(i > 0)
    def _():
        better = lmax > best_val[0]
        best_val[0] = jnp.where(better, lmax, best_val[0])
        best_idx[0] = jnp.where(better, lidx, best_idx[0])

    @pl.when(i == NB - 1)
    def _():
        out_ref[0] = best_idx[0]


def kernel(x, W, b):
    x0 = x[0:1, :]
    b2 = b.reshape(1, V)
    topk_id = pl.pallas_call(
        _top1_body,
        grid=(NB,),
        in_specs=[
            pl.BlockSpec((1, D), lambda i: (0, 0)),
            pl.BlockSpec((D, BV), lambda i: (0, i)),
            pl.BlockSpec((1, BV), lambda i: (0, i)),
        ],
        out_specs=pl.BlockSpec(memory_space=pltpu.SMEM),
        out_shape=jax.ShapeDtypeStruct((1,), jnp.int32),
        scratch_shapes=[
            pltpu.SMEM((1,), jnp.float32),
            pltpu.SMEM((1,), jnp.int32),
        ],
    )(x0, W, b2)
    return topk_id
